# baseline (speedup 1.0000x reference)
import jax
import jax.numpy as jnp
from jax.experimental import pallas as pl
from jax.experimental.pallas import tpu as pltpu
from jax.experimental.pallas import tpu_sc as plsc

C = 1024
H = 16
D = C // H
POS_DIM = 16
MAX_SAMPLES = 32
HALF_S = MAX_SAMPLES // 2
S = 2 * HALF_S + 1
MAX_FREQ = 16.0
MIN_FREQ = 1.0
SCALE = POS_DIM ** -0.5
L = 2048

PRE_BL = 256
POST_BL = 256
SP = 40

LS = 1024
G_BL = 128
G_W = 768

NC = 2
NS = 16
NW = NC * NS
NL = (L - LS) // NW


def _silu(v):
    return v * jax.nn.sigmoid(v)


def _pre_kernel(x_ref, wave_wT_ref, wave_b_ref, query_wT_ref, query_b_ref,
                kw_ref, attn_ref, idx_ref):
    i = pl.program_id(0)
    xb = x_ref[0]
    wave = _silu(jnp.dot(xb, wave_wT_ref[...], preferred_element_type=jnp.float32)
                 + wave_b_ref[...])
    queries = _silu(jnp.dot(xb, query_wT_ref[...], preferred_element_type=jnp.float32)
                    + query_b_ref[...])
    freq = jax.nn.sigmoid(wave[:, 0:H]) * (MAX_FREQ - MIN_FREQ) + MIN_FREQ
    phase = jnp.tanh(wave[:, H:2 * H]) * MAX_FREQ
    decay = jax.nn.sigmoid(wave[:, 2 * H:3 * H]) * 9.5 + 0.5
    freq_avg = jnp.mean(freq, axis=1, keepdims=True)
    phase_avg = jnp.mean(phase, axis=1, keepdims=True)
    decay_avg = jnp.mean(decay, axis=1, keepdims=True)
    qk = jnp.dot(queries, kw_ref[...], preferred_element_type=jnp.float32)

    stride = (jax.lax.broadcasted_iota(jnp.int32, (1, S), 1)
              - HALF_S).astype(jnp.float32)
    centers = (jax.lax.broadcasted_iota(jnp.int32, (PRE_BL, 1), 0)
               + i * PRE_BL).astype(jnp.float32)
    pos = centers + stride * freq_avg + phase_avg
    valid = (pos >= 0.0) & (pos < float(L))
    validf = valid.astype(jnp.float32)
    idx = jnp.clip(pos.astype(jnp.int32), 0, L - 1)
    rel = jnp.abs(stride) * freq_avg
    denv = jnp.exp(-rel / jnp.maximum(decay_avg, 0.1)) * validf
    relS = rel * SCALE

    for h in range(H):
        sc = qk[:, h:h + 1] * relS
        sc = jnp.where(valid, sc, -1e30)
        m = jnp.max(sc, axis=1, keepdims=True)
        e = jnp.exp(sc - m)
        a = e / jnp.sum(e, axis=1, keepdims=True)
        a = a * denv
        a = a / (jnp.sum(a, axis=1, keepdims=True) + 1e-8)
        attn_ref[:, h, 0:S] = a
    idx_ref[...] = jnp.concatenate(
        [idx, jnp.zeros((PRE_BL, SP - S), jnp.int32)], axis=1)


def _sc_gather_body(x_hbm, w_hbm, idxp_hbm, out_hbm,
                    idx_v, w_v, rows0, rows1, out_v, sem0, sem1):
    wid = jax.lax.axis_index("s") * NC + jax.lax.axis_index("c")
    obase = wid * NL
    base = LS + obase
    pltpu.sync_copy(idxp_hbm.at[pl.ds(base * SP, NL * SP)], idx_v)
    pltpu.sync_copy(w_hbm.at[pl.ds(base * H * SP, NL * H * SP)],
                    w_v.at[pl.ds(0, NL * H * SP)])

    def fire(li, rbuf, sem):
        pltpu.async_copy(x_hbm.at[idx_v.at[pl.ds(li * SP, S)]], rbuf, sem)

    def wait(li, rbuf, sem):
        pltpu.make_async_copy(
            x_hbm.at[idx_v.at[pl.ds(li * SP, S)]], rbuf, sem).wait()

    def compute(li, rbuf):
        dn = jax.lax.GatherDimensionNumbers(
            offset_dims=(), collapsed_slice_dims=(0,), start_index_map=(0,))

        def hbody(h, carry, rbuf=rbuf):
            woff = pl.multiple_of(li * (H * SP) + h * SP, 8)
            wr0 = w_v[pl.ds(woff, 16)]
            wr1 = w_v[pl.ds(woff + 16, 16)]
            wr2 = w_v[pl.ds(woff + 32, 16)]
            c0 = pl.multiple_of(h * D, 16)
            a0 = jnp.zeros((16,), jnp.float32)
            a1 = jnp.zeros((16,), jnp.float32)
            a2 = jnp.zeros((16,), jnp.float32)
            a3 = jnp.zeros((16,), jnp.float32)
            for s in range(S):
                wr = (wr0, wr1, wr2)[s // 16]
                wb = jax.lax.gather(
                    wr, jnp.full((16, 1), s % 16, jnp.int32), dn, (1,),
                    mode=jax.lax.GatherScatterMode.PROMISE_IN_BOUNDS)
                a0 = a0 + wb * rbuf[s, pl.ds(c0, 16)]
                a1 = a1 + wb * rbuf[s, pl.ds(c0 + 16, 16)]
                a2 = a2 + wb * rbuf[s, pl.ds(c0 + 32, 16)]
                a3 = a3 + wb * rbuf[s, pl.ds(c0 + 48, 16)]
            out_v[pl.ds(c0, 16)] = a0
            out_v[pl.ds(c0 + 16, 16)] = a1
            out_v[pl.ds(c0 + 32, 16)] = a2
            out_v[pl.ds(c0 + 48, 16)] = a3
            return carry

        jax.lax.fori_loop(0, H, hbody, 0)
        pltpu.sync_copy(out_v, out_hbm.at[pl.ds((obase + li) * C, C)])

    fire(0, rows0, sem0)

    def outer(g, carry):
        li0 = g * 2
        fire(li0 + 1, rows1, sem1)
        wait(li0, rows0, sem0)
        compute(li0, rows0)

        @pl.when(li0 + 2 < NL)
        def _():
            fire(li0 + 2, rows0, sem0)

        wait(li0 + 1, rows1, sem1)
        compute(li0 + 1, rows1)
        return carry

    jax.lax.fori_loop(0, NL // 2, outer, 0)


def _tc_gather_kernel(x_ref, attn_ref, idx_ref, out_ref):
    i = pl.program_id(0)
    l0 = i * G_BL
    w0 = pl.multiple_of(jnp.clip(l0 - 272, 0, L - G_W), 8)
    xw = x_ref[pl.ds(w0, G_W), :]
    lane = jax.lax.broadcasted_iota(jnp.int32, (G_BL, G_W), 1)
    hsel = jax.lax.broadcasted_iota(jnp.int32, (H, C), 1) // D
    hrow = jax.lax.broadcasted_iota(jnp.int32, (H, C), 0)
    expand = (hsel == hrow).astype(jnp.float32)
    acc = jnp.zeros((G_BL, C), jnp.float32)
    for s in range(S):
        rel_idx = idx_ref[:, s:s + 1] - w0
        p = (rel_idx == lane).astype(jnp.bfloat16)
        ws = jnp.dot(attn_ref[:, s, :], expand,
                     preferred_element_type=jnp.float32)
        acc = acc + jnp.dot(p, xw, preferred_element_type=jnp.float32) * ws
    out_ref[...] = acc


def _post_kernel(o_ref, se1_wT_ref, se1_b_ref, se2_wT_ref, se2_b_ref,
                 out_wT_ref, out_ref):
    o = o_ref[...]
    h1 = _silu(jnp.dot(o, se1_wT_ref[...], preferred_element_type=jnp.float32)
               + se1_b_ref[...])
    se = jax.nn.sigmoid(jnp.dot(h1, se2_wT_ref[...], preferred_element_type=jnp.float32)
                        + se2_b_ref[...])
    g = o * se
    out_ref[0] = _silu(jnp.dot(g, out_wT_ref[...], preferred_element_type=jnp.float32))


@jax.jit
def kernel(x, wave_w, wave_b, query_w, query_b, key_weight, out_w, se1_w,
           se1_b, se2_w, se2_b):
    B = x.shape[0]
    x2 = x.reshape(L, C)
    hmask = jnp.repeat(jnp.eye(H, dtype=jnp.float32), POS_DIM, axis=0)
    kw_mat = hmask * jnp.tile(key_weight, H)[:, None]

    n_pre = L // PRE_BL
    attn, idxp = pl.pallas_call(
        _pre_kernel,
        grid=(n_pre,),
        in_specs=[
            pl.BlockSpec((1, PRE_BL, C), lambda i: (0, i, 0)),
            pl.BlockSpec((C, 3 * H), lambda i: (0, 0)),
            pl.BlockSpec((1, 3 * H), lambda i: (0, 0)),
            pl.BlockSpec((C, H * POS_DIM), lambda i: (0, 0)),
            pl.BlockSpec((1, H * POS_DIM), lambda i: (0, 0)),
            pl.BlockSpec((H * POS_DIM, H), lambda i: (0, 0)),
        ],
        out_specs=[
            pl.BlockSpec((PRE_BL, H, SP), lambda i: (i, 0, 0)),
            pl.BlockSpec((PRE_BL, SP), lambda i: (i, 0)),
        ],
        out_shape=[
            jax.ShapeDtypeStruct((L, H, SP), jnp.float32),
            jax.ShapeDtypeStruct((L, SP), jnp.int32),
        ],
    )(x, wave_w.T, wave_b[None], query_w.T, query_b[None], kw_mat)

    attn_flat = attn.reshape(L * H * SP)
    idxp_flat = idxp.reshape(L * SP)

    sc_gather = pl.kernel(
        _sc_gather_body,
        out_type=jax.ShapeDtypeStruct(((L - LS) * C,), jnp.float32),
        mesh=plsc.VectorSubcoreMesh(core_axis_name="c", subcore_axis_name="s",
                                    num_cores=NC, num_subcores=NS),
        scratch_types=[
            pltpu.VMEM((NL * SP,), jnp.int32),
            pltpu.VMEM((NL * H * SP + 16,), jnp.float32),
            pltpu.VMEM((S, C), jnp.float32),
            pltpu.VMEM((S, C), jnp.float32),
            pltpu.VMEM((C,), jnp.float32),
            pltpu.SemaphoreType.DMA,
            pltpu.SemaphoreType.DMA,
        ],
    )
    out_sc = sc_gather(x2, attn_flat, idxp_flat).reshape(L - LS, C)

    attn_t = attn[:LS].transpose(0, 2, 1)
    xb16 = x2.astype(jnp.bfloat16)
    n_g = LS // G_BL
    out_tc = pl.pallas_call(
        _tc_gather_kernel,
        grid=(n_g,),
        in_specs=[
            pl.BlockSpec((L, C), lambda i: (0, 0)),
            pl.BlockSpec((G_BL, SP, H), lambda i: (i, 0, 0)),
            pl.BlockSpec((G_BL, SP), lambda i: (i, 0)),
        ],
        out_specs=pl.BlockSpec((G_BL, C), lambda i: (i, 0)),
        out_shape=jax.ShapeDtypeStruct((LS, C), jnp.float32),
    )(xb16, attn_t, idxp[:LS])

    out1 = jnp.concatenate([out_tc, out_sc], axis=0)

    n_post = L // POST_BL
    out = pl.pallas_call(
        _post_kernel,
        grid=(n_post,),
        in_specs=[
            pl.BlockSpec((POST_BL, C), lambda i: (i, 0)),
            pl.BlockSpec((C, C // 4), lambda i: (0, 0)),
            pl.BlockSpec((1, C // 4), lambda i: (0, 0)),
            pl.BlockSpec((C // 4, C), lambda i: (0, 0)),
            pl.BlockSpec((1, C), lambda i: (0, 0)),
            pl.BlockSpec((C, C), lambda i: (0, 0)),
        ],
        out_specs=pl.BlockSpec((1, POST_BL, C), lambda i: (0, i, 0)),
        out_shape=jax.ShapeDtypeStruct((B, L, C), jnp.float32),
    )(out1, se1_w.T, se1_b[None], se2_w.T, se2_b[None], out_w.T)
    return out


# speedup vs baseline: 1.2320x; 1.2320x over previous
import jax
import jax.numpy as jnp
from jax.experimental import pallas as pl
from jax.experimental.pallas import tpu as pltpu
from jax.experimental.pallas import tpu_sc as plsc

C = 1024
H = 16
D = C // H
POS_DIM = 16
MAX_SAMPLES = 32
HALF_S = MAX_SAMPLES // 2
S = 2 * HALF_S + 1
MAX_FREQ = 16.0
MIN_FREQ = 1.0
SCALE = POS_DIM ** -0.5
L = 2048

PRE_BL = 256
POST_BL = 256
SP = 40

LS = 896
G_BL = 128
G_W = 768

NC = 2
NS = 16
NW = NC * NS
NL = (L - LS) // NW


def _silu(v):
    return v * jax.nn.sigmoid(v)


def _pre_kernel(x_ref, wave_wT_ref, wave_b_ref, query_wT_ref, query_b_ref,
                kw_ref, attn_ref, idx_ref):
    i = pl.program_id(0)
    xb = x_ref[0]
    wave = _silu(jnp.dot(xb, wave_wT_ref[...], preferred_element_type=jnp.float32)
                 + wave_b_ref[...])
    queries = _silu(jnp.dot(xb, query_wT_ref[...], preferred_element_type=jnp.float32)
                    + query_b_ref[...])
    freq = jax.nn.sigmoid(wave[:, 0:H]) * (MAX_FREQ - MIN_FREQ) + MIN_FREQ
    phase = jnp.tanh(wave[:, H:2 * H]) * MAX_FREQ
    decay = jax.nn.sigmoid(wave[:, 2 * H:3 * H]) * 9.5 + 0.5
    freq_avg = jnp.mean(freq, axis=1, keepdims=True)
    phase_avg = jnp.mean(phase, axis=1, keepdims=True)
    decay_avg = jnp.mean(decay, axis=1, keepdims=True)
    qk = jnp.dot(queries, kw_ref[...], preferred_element_type=jnp.float32)

    stride = (jax.lax.broadcasted_iota(jnp.int32, (1, S), 1)
              - HALF_S).astype(jnp.float32)
    centers = (jax.lax.broadcasted_iota(jnp.int32, (PRE_BL, 1), 0)
               + i * PRE_BL).astype(jnp.float32)
    pos = centers + stride * freq_avg + phase_avg
    valid = (pos >= 0.0) & (pos < float(L))
    validf = valid.astype(jnp.float32)
    idx = jnp.clip(pos.astype(jnp.int32), 0, L - 1)
    rel = jnp.abs(stride) * freq_avg
    denv = jnp.exp(-rel / jnp.maximum(decay_avg, 0.1)) * validf
    relS = rel * SCALE

    for h in range(H):
        sc = qk[:, h:h + 1] * relS
        sc = jnp.where(valid, sc, -1e30)
        m = jnp.max(sc, axis=1, keepdims=True)
        e = jnp.exp(sc - m)
        a = e / jnp.sum(e, axis=1, keepdims=True)
        a = a * denv
        a = a / (jnp.sum(a, axis=1, keepdims=True) + 1e-8)
        attn_ref[:, h, 0:S] = a
    idx_ref[...] = jnp.concatenate(
        [idx, jnp.zeros((PRE_BL, SP - S), jnp.int32)], axis=1)


def _sc_gather_body(x_hbm, w_hbm, idxp_hbm, out_hbm,
                    idx_v, w_v, rows0, rows1, out_v, sem0, sem1):
    wid = jax.lax.axis_index("s") * NC + jax.lax.axis_index("c")
    obase = wid * NL
    base = LS + obase
    pltpu.sync_copy(idxp_hbm.at[pl.ds(base * SP, NL * SP)], idx_v)
    pltpu.sync_copy(w_hbm.at[pl.ds(base * H * SP, NL * H * SP)],
                    w_v.at[pl.ds(0, NL * H * SP)])

    def fire(li, rbuf, sem):
        pltpu.async_copy(x_hbm.at[idx_v.at[pl.ds(li * SP, S)]], rbuf, sem)

    def wait(li, rbuf, sem):
        pltpu.make_async_copy(
            x_hbm.at[idx_v.at[pl.ds(li * SP, S)]], rbuf, sem).wait()

    def compute(li, rbuf):
        dn = jax.lax.GatherDimensionNumbers(
            offset_dims=(), collapsed_slice_dims=(0,), start_index_map=(0,))

        def hbody(h, carry, rbuf=rbuf):
            woff = pl.multiple_of(li * (H * SP) + h * SP, 8)
            wr0 = w_v[pl.ds(woff, 16)]
            wr1 = w_v[pl.ds(woff + 16, 16)]
            wr2 = w_v[pl.ds(woff + 32, 16)]
            c0 = pl.multiple_of(h * D, 16)
            a0 = jnp.zeros((16,), jnp.float32)
            a1 = jnp.zeros((16,), jnp.float32)
            a2 = jnp.zeros((16,), jnp.float32)
            a3 = jnp.zeros((16,), jnp.float32)
            for s in range(S):
                wr = (wr0, wr1, wr2)[s // 16]
                wb = jax.lax.gather(
                    wr, jnp.full((16, 1), s % 16, jnp.int32), dn, (1,),
                    mode=jax.lax.GatherScatterMode.PROMISE_IN_BOUNDS)
                a0 = a0 + wb * rbuf[s, pl.ds(c0, 16)]
                a1 = a1 + wb * rbuf[s, pl.ds(c0 + 16, 16)]
                a2 = a2 + wb * rbuf[s, pl.ds(c0 + 32, 16)]
                a3 = a3 + wb * rbuf[s, pl.ds(c0 + 48, 16)]
            out_v[pl.ds(c0, 16)] = a0
            out_v[pl.ds(c0 + 16, 16)] = a1
            out_v[pl.ds(c0 + 32, 16)] = a2
            out_v[pl.ds(c0 + 48, 16)] = a3
            return carry

        jax.lax.fori_loop(0, H, hbody, 0)
        pltpu.sync_copy(out_v, out_hbm.at[pl.ds((obase + li) * C, C)])

    fire(0, rows0, sem0)

    def outer(g, carry):
        li0 = g * 2
        fire(li0 + 1, rows1, sem1)
        wait(li0, rows0, sem0)
        compute(li0, rows0)

        @pl.when(li0 + 2 < NL)
        def _():
            fire(li0 + 2, rows0, sem0)

        wait(li0 + 1, rows1, sem1)
        compute(li0 + 1, rows1)
        return carry

    jax.lax.fori_loop(0, NL // 2, outer, 0)


def _tc_gather_kernel(x_ref, attn_ref, idx_ref, out_ref):
    i = pl.program_id(0)
    l0 = i * G_BL
    w0 = pl.multiple_of(jnp.clip(l0 - 272, 0, L - G_W), 8)
    xw = x_ref[pl.ds(w0, G_W), :].astype(jnp.bfloat16)
    lane = jax.lax.broadcasted_iota(jnp.int32, (G_BL, G_W), 1)
    hsel = jax.lax.broadcasted_iota(jnp.int32, (H, C), 1) // D
    hrow = jax.lax.broadcasted_iota(jnp.int32, (H, C), 0)
    expand = (hsel == hrow).astype(jnp.float32)
    at2 = jnp.swapaxes(attn_ref[...], 1, 2)
    acc = jnp.zeros((G_BL, C), jnp.float32)
    for s in range(S):
        rel_idx = idx_ref[:, s:s + 1] - w0
        p = (rel_idx == lane).astype(jnp.bfloat16)
        ws = jnp.dot(at2[:, s, :], expand,
                     preferred_element_type=jnp.float32)
        acc = acc + jnp.dot(p, xw, preferred_element_type=jnp.float32) * ws
    out_ref[...] = acc


def _post_kernel(o_ref, se1_wT_ref, se1_b_ref, se2_wT_ref, se2_b_ref,
                 out_wT_ref, out_ref):
    o = o_ref[...]
    h1 = _silu(jnp.dot(o, se1_wT_ref[...], preferred_element_type=jnp.float32)
               + se1_b_ref[...])
    se = jax.nn.sigmoid(jnp.dot(h1, se2_wT_ref[...], preferred_element_type=jnp.float32)
                        + se2_b_ref[...])
    g = o * se
    out_ref[0] = _silu(jnp.dot(g, out_wT_ref[...], preferred_element_type=jnp.float32))


@jax.jit
def kernel(x, wave_w, wave_b, query_w, query_b, key_weight, out_w, se1_w,
           se1_b, se2_w, se2_b):
    B = x.shape[0]
    x2 = x.reshape(L, C)
    hmask = jnp.repeat(jnp.eye(H, dtype=jnp.float32), POS_DIM, axis=0)
    kw_mat = hmask * jnp.tile(key_weight, H)[:, None]

    n_pre = L // PRE_BL
    attn, idxp = pl.pallas_call(
        _pre_kernel,
        grid=(n_pre,),
        in_specs=[
            pl.BlockSpec((1, PRE_BL, C), lambda i: (0, i, 0)),
            pl.BlockSpec((C, 3 * H), lambda i: (0, 0)),
            pl.BlockSpec((1, 3 * H), lambda i: (0, 0)),
            pl.BlockSpec((C, H * POS_DIM), lambda i: (0, 0)),
            pl.BlockSpec((1, H * POS_DIM), lambda i: (0, 0)),
            pl.BlockSpec((H * POS_DIM, H), lambda i: (0, 0)),
        ],
        out_specs=[
            pl.BlockSpec((PRE_BL, H, SP), lambda i: (i, 0, 0)),
            pl.BlockSpec((PRE_BL, SP), lambda i: (i, 0)),
        ],
        out_shape=[
            jax.ShapeDtypeStruct((L, H, SP), jnp.float32),
            jax.ShapeDtypeStruct((L, SP), jnp.int32),
        ],
    )(x, wave_w.T, wave_b[None], query_w.T, query_b[None], kw_mat)

    attn_flat = attn.reshape(L * H * SP)
    idxp_flat = idxp.reshape(L * SP)

    sc_gather = pl.kernel(
        _sc_gather_body,
        out_type=jax.ShapeDtypeStruct(((L - LS) * C,), jnp.float32),
        mesh=plsc.VectorSubcoreMesh(core_axis_name="c", subcore_axis_name="s",
                                    num_cores=NC, num_subcores=NS),
        scratch_types=[
            pltpu.VMEM((NL * SP,), jnp.int32),
            pltpu.VMEM((NL * H * SP + 16,), jnp.float32),
            pltpu.VMEM((S, C), jnp.float32),
            pltpu.VMEM((S, C), jnp.float32),
            pltpu.VMEM((C,), jnp.float32),
            pltpu.SemaphoreType.DMA,
            pltpu.SemaphoreType.DMA,
        ],
    )
    out_sc = sc_gather(x2, attn_flat, idxp_flat).reshape(L - LS, C)

    n_g = LS // G_BL
    out_tc = pl.pallas_call(
        _tc_gather_kernel,
        grid=(n_g,),
        in_specs=[
            pl.BlockSpec((L, C), lambda i: (0, 0)),
            pl.BlockSpec((G_BL, H, SP), lambda i: (i, 0, 0)),
            pl.BlockSpec((G_BL, SP), lambda i: (i, 0)),
        ],
        out_specs=pl.BlockSpec((G_BL, C), lambda i: (i, 0)),
        out_shape=jax.ShapeDtypeStruct((LS, C), jnp.float32),
    )(x2, attn, idxp)

    out1 = jnp.concatenate([out_tc, out_sc], axis=0)

    n_post = L // POST_BL
    out = pl.pallas_call(
        _post_kernel,
        grid=(n_post,),
        in_specs=[
            pl.BlockSpec((POST_BL, C), lambda i: (i, 0)),
            pl.BlockSpec((C, C // 4), lambda i: (0, 0)),
            pl.BlockSpec((1, C // 4), lambda i: (0, 0)),
            pl.BlockSpec((C // 4, C), lambda i: (0, 0)),
            pl.BlockSpec((1, C), lambda i: (0, 0)),
            pl.BlockSpec((C, C), lambda i: (0, 0)),
        ],
        out_specs=pl.BlockSpec((1, POST_BL, C), lambda i: (0, i, 0)),
        out_shape=jax.ShapeDtypeStruct((B, L, C), jnp.float32),
    )(out1, se1_w.T, se1_b[None], se2_w.T, se2_b[None], out_w.T)
    return out


# speedup vs baseline: 1.3041x; 1.0585x over previous
import jax
import jax.numpy as jnp
from jax.experimental import pallas as pl
from jax.experimental.pallas import tpu as pltpu
from jax.experimental.pallas import tpu_sc as plsc

C = 1024
H = 16
D = C // H
POS_DIM = 16
MAX_SAMPLES = 32
HALF_S = MAX_SAMPLES // 2
S = 2 * HALF_S + 1
MAX_FREQ = 16.0
MIN_FREQ = 1.0
SCALE = POS_DIM ** -0.5
L = 2048

PRE_BL = 256
POST_BL = 256
SP = 40

LS = 1152
G_BL = 128
G_W = 768

NC = 2
NS = 16
NW = NC * NS
NL = (L - LS) // NW


def _silu(v):
    return v * jax.nn.sigmoid(v)


def _pre_kernel(x_ref, wave_wT_ref, wave_b_ref, query_wT_ref, query_b_ref,
                kw_ref, attn_ref, idx_ref):
    i = pl.program_id(0)
    xb = x_ref[0]
    wave = _silu(jnp.dot(xb, wave_wT_ref[...], preferred_element_type=jnp.float32)
                 + wave_b_ref[...])
    queries = _silu(jnp.dot(xb, query_wT_ref[...], preferred_element_type=jnp.float32)
                    + query_b_ref[...])
    freq = jax.nn.sigmoid(wave[:, 0:H]) * (MAX_FREQ - MIN_FREQ) + MIN_FREQ
    phase = jnp.tanh(wave[:, H:2 * H]) * MAX_FREQ
    decay = jax.nn.sigmoid(wave[:, 2 * H:3 * H]) * 9.5 + 0.5
    freq_avg = jnp.mean(freq, axis=1, keepdims=True)
    phase_avg = jnp.mean(phase, axis=1, keepdims=True)
    decay_avg = jnp.mean(decay, axis=1, keepdims=True)
    qk = jnp.dot(queries, kw_ref[...], preferred_element_type=jnp.float32)

    stride = (jax.lax.broadcasted_iota(jnp.int32, (1, S), 1)
              - HALF_S).astype(jnp.float32)
    centers = (jax.lax.broadcasted_iota(jnp.int32, (PRE_BL, 1), 0)
               + i * PRE_BL).astype(jnp.float32)
    pos = centers + stride * freq_avg + phase_avg
    valid = (pos >= 0.0) & (pos < float(L))
    validf = valid.astype(jnp.float32)
    idx = jnp.clip(pos.astype(jnp.int32), 0, L - 1)
    rel = jnp.abs(stride) * freq_avg
    denv = jnp.exp(-rel / jnp.maximum(decay_avg, 0.1)) * validf
    relS = rel * SCALE

    for h in range(H):
        sc = qk[:, h:h + 1] * relS
        sc = jnp.where(valid, sc, -1e30)
        m = jnp.max(sc, axis=1, keepdims=True)
        e = jnp.exp(sc - m)
        a = e / jnp.sum(e, axis=1, keepdims=True)
        a = a * denv
        a = a / (jnp.sum(a, axis=1, keepdims=True) + 1e-8)
        attn_ref[:, h, 0:S] = a
    idx_ref[...] = jnp.concatenate(
        [idx, jnp.zeros((PRE_BL, SP - S), jnp.int32)], axis=1)


def _sc_gather_body(x_hbm, w_hbm, idxp_hbm, out_hbm,
                    idx_v, w_v, rows0, rows1, out_v, sem0, sem1):
    wid = jax.lax.axis_index("s") * NC + jax.lax.axis_index("c")
    obase = wid * NL
    base = LS + obase
    pltpu.sync_copy(idxp_hbm.at[pl.ds(base * SP, NL * SP)], idx_v)
    pltpu.sync_copy(w_hbm.at[pl.ds(base * H * SP, NL * H * SP)],
                    w_v.at[pl.ds(0, NL * H * SP)])

    def fire(li, rbuf, sem):
        pltpu.async_copy(x_hbm.at[idx_v.at[pl.ds(li * SP, S)]], rbuf, sem)

    def wait(li, rbuf, sem):
        pltpu.make_async_copy(
            x_hbm.at[idx_v.at[pl.ds(li * SP, S)]], rbuf, sem).wait()

    def compute(li, rbuf):
        dn = jax.lax.GatherDimensionNumbers(
            offset_dims=(), collapsed_slice_dims=(0,), start_index_map=(0,))

        def hbody(h, carry, rbuf=rbuf):
            woff = pl.multiple_of(li * (H * SP) + h * SP, 8)
            wr0 = w_v[pl.ds(woff, 16)]
            wr1 = w_v[pl.ds(woff + 16, 16)]
            wr2 = w_v[pl.ds(woff + 32, 16)]
            c0 = pl.multiple_of(h * D, 16)
            a0 = jnp.zeros((16,), jnp.float32)
            a1 = jnp.zeros((16,), jnp.float32)
            a2 = jnp.zeros((16,), jnp.float32)
            a3 = jnp.zeros((16,), jnp.float32)
            for s in range(S):
                wr = (wr0, wr1, wr2)[s // 16]
                wb = jax.lax.gather(
                    wr, jnp.full((16, 1), s % 16, jnp.int32), dn, (1,),
                    mode=jax.lax.GatherScatterMode.PROMISE_IN_BOUNDS)
                a0 = a0 + wb * rbuf[s, pl.ds(c0, 16)]
                a1 = a1 + wb * rbuf[s, pl.ds(c0 + 16, 16)]
                a2 = a2 + wb * rbuf[s, pl.ds(c0 + 32, 16)]
                a3 = a3 + wb * rbuf[s, pl.ds(c0 + 48, 16)]
            out_v[pl.ds(c0, 16)] = a0
            out_v[pl.ds(c0 + 16, 16)] = a1
            out_v[pl.ds(c0 + 32, 16)] = a2
            out_v[pl.ds(c0 + 48, 16)] = a3
            return carry

        jax.lax.fori_loop(0, H, hbody, 0)
        pltpu.sync_copy(out_v, out_hbm.at[pl.ds((obase + li) * C, C)])

    fire(0, rows0, sem0)

    def outer(g, carry):
        li0 = g * 2
        fire(li0 + 1, rows1, sem1)
        wait(li0, rows0, sem0)
        compute(li0, rows0)

        @pl.when(li0 + 2 < NL)
        def _():
            fire(li0 + 2, rows0, sem0)

        wait(li0 + 1, rows1, sem1)
        compute(li0 + 1, rows1)
        return carry

    jax.lax.fori_loop(0, NL // 2, outer, 0)


def _tc_gather_kernel(x_ref, attn_ref, idx_ref, out_ref):
    i = pl.program_id(0)
    l0 = i * G_BL
    w0 = pl.multiple_of(jnp.clip(l0 - 272, 0, L - G_W), 8)
    xw = x_ref[pl.ds(w0, G_W), :].astype(jnp.bfloat16)
    lane = jax.lax.broadcasted_iota(jnp.int32, (G_BL, G_W), 1)
    hsel = jax.lax.broadcasted_iota(jnp.int32, (H, C), 1) // D
    hrow = jax.lax.broadcasted_iota(jnp.int32, (H, C), 0)
    expand = (hsel == hrow).astype(jnp.float32)
    at2 = jnp.swapaxes(attn_ref[...], 1, 2)
    acc = jnp.zeros((G_BL, C), jnp.float32)
    for s in range(S):
        rel_idx = idx_ref[:, s:s + 1] - w0
        p = (rel_idx == lane).astype(jnp.bfloat16)
        ws = jnp.dot(at2[:, s, :], expand,
                     preferred_element_type=jnp.float32)
        acc = acc + jnp.dot(p, xw, preferred_element_type=jnp.float32) * ws
    out_ref[...] = acc


def _post_kernel(o_ref, se1_wT_ref, se1_b_ref, se2_wT_ref, se2_b_ref,
                 out_wT_ref, out_ref):
    o = o_ref[...]
    h1 = _silu(jnp.dot(o, se1_wT_ref[...], preferred_element_type=jnp.float32)
               + se1_b_ref[...])
    se = jax.nn.sigmoid(jnp.dot(h1, se2_wT_ref[...], preferred_element_type=jnp.float32)
                        + se2_b_ref[...])
    g = o * se
    out_ref[0] = _silu(jnp.dot(g, out_wT_ref[...], preferred_element_type=jnp.float32))


@jax.jit
def kernel(x, wave_w, wave_b, query_w, query_b, key_weight, out_w, se1_w,
           se1_b, se2_w, se2_b):
    B = x.shape[0]
    x2 = x.reshape(L, C)
    hmask = jnp.repeat(jnp.eye(H, dtype=jnp.float32), POS_DIM, axis=0)
    kw_mat = hmask * jnp.tile(key_weight, H)[:, None]

    n_pre = L // PRE_BL
    attn, idxp = pl.pallas_call(
        _pre_kernel,
        grid=(n_pre,),
        in_specs=[
            pl.BlockSpec((1, PRE_BL, C), lambda i: (0, i, 0)),
            pl.BlockSpec((C, 3 * H), lambda i: (0, 0)),
            pl.BlockSpec((1, 3 * H), lambda i: (0, 0)),
            pl.BlockSpec((C, H * POS_DIM), lambda i: (0, 0)),
            pl.BlockSpec((1, H * POS_DIM), lambda i: (0, 0)),
            pl.BlockSpec((H * POS_DIM, H), lambda i: (0, 0)),
        ],
        out_specs=[
            pl.BlockSpec((PRE_BL, H, SP), lambda i: (i, 0, 0)),
            pl.BlockSpec((PRE_BL, SP), lambda i: (i, 0)),
        ],
        out_shape=[
            jax.ShapeDtypeStruct((L, H, SP), jnp.float32),
            jax.ShapeDtypeStruct((L, SP), jnp.int32),
        ],
    )(x, wave_w.T, wave_b[None], query_w.T, query_b[None], kw_mat)

    attn_flat = attn.reshape(L * H * SP)
    idxp_flat = idxp.reshape(L * SP)

    sc_gather = pl.kernel(
        _sc_gather_body,
        out_type=jax.ShapeDtypeStruct(((L - LS) * C,), jnp.float32),
        mesh=plsc.VectorSubcoreMesh(core_axis_name="c", subcore_axis_name="s",
                                    num_cores=NC, num_subcores=NS),
        scratch_types=[
            pltpu.VMEM((NL * SP,), jnp.int32),
            pltpu.VMEM((NL * H * SP + 16,), jnp.float32),
            pltpu.VMEM((S, C), jnp.float32),
            pltpu.VMEM((S, C), jnp.float32),
            pltpu.VMEM((C,), jnp.float32),
            pltpu.SemaphoreType.DMA,
            pltpu.SemaphoreType.DMA,
        ],
    )
    out_sc = sc_gather(x2, attn_flat, idxp_flat).reshape(L - LS, C)

    n_g = LS // G_BL
    out_tc = pl.pallas_call(
        _tc_gather_kernel,
        grid=(n_g,),
        in_specs=[
            pl.BlockSpec((L, C), lambda i: (0, 0)),
            pl.BlockSpec((G_BL, H, SP), lambda i: (i, 0, 0)),
            pl.BlockSpec((G_BL, SP), lambda i: (i, 0)),
        ],
        out_specs=pl.BlockSpec((G_BL, C), lambda i: (i, 0)),
        out_shape=jax.ShapeDtypeStruct((LS, C), jnp.float32),
    )(x2, attn, idxp)

    out1 = jnp.concatenate([out_tc, out_sc], axis=0)

    n_post = L // POST_BL
    out = pl.pallas_call(
        _post_kernel,
        grid=(n_post,),
        in_specs=[
            pl.BlockSpec((POST_BL, C), lambda i: (i, 0)),
            pl.BlockSpec((C, C // 4), lambda i: (0, 0)),
            pl.BlockSpec((1, C // 4), lambda i: (0, 0)),
            pl.BlockSpec((C // 4, C), lambda i: (0, 0)),
            pl.BlockSpec((1, C), lambda i: (0, 0)),
            pl.BlockSpec((C, C), lambda i: (0, 0)),
        ],
        out_specs=pl.BlockSpec((1, POST_BL, C), lambda i: (0, i, 0)),
        out_shape=jax.ShapeDtypeStruct((B, L, C), jnp.float32),
    )(out1, se1_w.T, se1_b[None], se2_w.T, se2_b[None], out_w.T)
    return out
